# baseline (speedup 1.0000x reference)
import functools

import jax
import jax.numpy as jnp
from jax import lax
from jax.experimental import pallas as pl
from jax.experimental.pallas import tpu as pltpu
from jax.experimental.pallas import tpu_sc as plsc

N = 10000
E = 320000
D = 128
PE = 16
NC = 2
NT = 16
K = 128
NCHUNK = 80
EPT = NCHUNK * K
EPAD = NC * NT * EPT
NROW = N + 128
RCH = 80
NRC = N // RCH

ROW_BLOCK = 2000
GRID = N // ROW_BLOCK


def _mm_body(x_ref, w_ref, t_ref):
    t_ref[...] = jnp.dot(x_ref[...], w_ref[...],
                         preferred_element_type=jnp.float32)


def _mm(x, w):
    return pl.pallas_call(
        _mm_body,
        grid=(GRID,),
        in_specs=[
            pl.BlockSpec((ROW_BLOCK, D), lambda i: (i, 0)),
            pl.BlockSpec((D, D), lambda i: (0, 0)),
        ],
        out_specs=pl.BlockSpec((ROW_BLOCK, D), lambda i: (i, 0)),
        out_shape=jax.ShapeDtypeStruct((N, D), jnp.float32),
    )(x, w)


def _epi_mm_body(a0_ref, a1_ref, t_ref, b_ref, w_ref, h_ref, u_ref):
    h = a0_ref[...] + a1_ref[...] - t_ref[...] + b_ref[...]
    h = jnp.maximum(h, 0.0)
    h_ref[...] = h
    u_ref[...] = jnp.dot(h, w_ref[...], preferred_element_type=jnp.float32)


def _epi_mm(accs, t, b, w):
    return pl.pallas_call(
        _epi_mm_body,
        grid=(GRID,),
        in_specs=[
            pl.BlockSpec((ROW_BLOCK, D), lambda i: (i, 0)),
            pl.BlockSpec((ROW_BLOCK, D), lambda i: (i + GRID, 0)),
            pl.BlockSpec((ROW_BLOCK, D), lambda i: (i, 0)),
            pl.BlockSpec((1, D), lambda i: (0, 0)),
            pl.BlockSpec((D, D), lambda i: (0, 0)),
        ],
        out_specs=[
            pl.BlockSpec((ROW_BLOCK, D), lambda i: (i, 0)),
            pl.BlockSpec((ROW_BLOCK, D), lambda i: (i, 0)),
        ],
        out_shape=[
            jax.ShapeDtypeStruct((N, D), jnp.float32),
            jax.ShapeDtypeStruct((N, D), jnp.float32),
        ],
    )(accs, accs, t, b, w)


def _final_body(a0_ref, a1_ref, t_ref, b_ref, h1_ref, wout_ref, bout_ref,
                out_ref):
    x2 = a0_ref[...] + a1_ref[...] - t_ref[...] + b_ref[...]
    x2 = jnp.maximum(x2, 0.0) + h1_ref[...]
    out_ref[...] = (
        jnp.dot(x2, wout_ref[...], preferred_element_type=jnp.float32)
        + bout_ref[...]
    )


def _final(accs, t, b, h1, wout, bout):
    return pl.pallas_call(
        _final_body,
        grid=(GRID,),
        in_specs=[
            pl.BlockSpec((ROW_BLOCK, D), lambda i: (i, 0)),
            pl.BlockSpec((ROW_BLOCK, D), lambda i: (i + GRID, 0)),
            pl.BlockSpec((ROW_BLOCK, D), lambda i: (i, 0)),
            pl.BlockSpec((1, D), lambda i: (0, 0)),
            pl.BlockSpec((ROW_BLOCK, D), lambda i: (i, 0)),
            pl.BlockSpec((D, PE), lambda i: (0, 0)),
            pl.BlockSpec((1, PE), lambda i: (0, 0)),
        ],
        out_specs=pl.BlockSpec((ROW_BLOCK, PE), lambda i: (i, 0)),
        out_shape=jax.ShapeDtypeStruct((N, PE), jnp.float32),
    )(accs, accs, t, b, h1, wout, bout)


def _agg_body(t_hbm, src_hbm, dst_hbm, accs_out,
              src_b, dst_b, rows_v, acc_sh,
              sem_g0, sem_g1, sem_s0, sem_s1,
              sem_i0, sem_i1, sem_i2, sem_i3):
    c = lax.axis_index("c")
    s = lax.axis_index("s")
    base = (c * NT + s) * NCHUNK
    sems_g = (sem_g0, sem_g1)
    sems_s = (sem_s0, sem_s1)
    sems_i = (sem_i0, sem_i1, sem_i2, sem_i3)
    nmine = (NRC - 1 - s) // NT + 1

    def icopies(j, q):
        return (pltpu.make_async_copy(src_hbm.at[base + j], src_b.at[q],
                                      sems_i[q]),
                pltpu.make_async_copy(dst_hbm.at[base + j], dst_b.at[q],
                                      sems_i[q]))

    def istart(j, q):
        ca, cb = icopies(j, q)
        ca.start()
        cb.start()

    def iwait(j, q):
        ca, cb = icopies(j, q)
        ca.wait()
        cb.wait()

    def gcopy(b, q):
        return pltpu.make_async_copy(
            t_hbm.at[src_b.at[q]], rows_v.at[b], sems_g[b])

    def swait(b, q):
        pltpu.make_async_copy(
            rows_v.at[b], acc_sh.at[dst_b.at[q]], sems_s[b]).wait()

    istart(0, 0)
    istart(1, 1)

    def init_body(q, carry):
        r = (s + q * NT) * RCH
        pltpu.sync_copy(t_hbm.at[pl.ds(r, RCH)], acc_sh.at[pl.ds(r, RCH)])
        return carry

    lax.fori_loop(0, nmine, init_body, 0)
    plsc.subcore_barrier()

    def step(j, r):
        b = r % 2
        gcopy(b, r).wait()
        pltpu.async_copy(rows_v.at[b], acc_sh.at[dst_b.at[r]],
                         sems_s[b], add=True)

        @pl.when(j + 2 < NCHUNK)
        def _():
            istart(j + 2, (r + 2) % 4)

        @pl.when(j >= 1)
        def _():
            swait(1 - b, (r + 3) % 4)

        @pl.when(j + 1 < NCHUNK)
        def _():
            iwait(j + 1, (r + 1) % 4)
            gcopy(1 - b, (r + 1) % 4).start()

    iwait(0, 0)
    gcopy(0, 0).start()

    def loop_body(i, carry):
        for r in range(4):
            step(i * 4 + r, r)
        return carry

    lax.fori_loop(0, NCHUNK // 4, loop_body, 0)
    swait(1, 3)
    plsc.subcore_barrier()

    def drain_body(q, carry):
        r = (s + q * NT) * RCH
        pltpu.sync_copy(acc_sh.at[pl.ds(r, RCH)],
                        accs_out.at[pl.ds(c * N + r, RCH)])
        return carry

    lax.fori_loop(0, nmine, drain_body, 0)


_agg = functools.partial(
    pl.kernel,
    mesh=plsc.VectorSubcoreMesh(core_axis_name="c", subcore_axis_name="s"),
    out_type=jax.ShapeDtypeStruct((NC * N, D), jnp.float32),
    scratch_types=[
        pltpu.VMEM((4, K), jnp.int32),
        pltpu.VMEM((4, K), jnp.int32),
        pltpu.VMEM((2, K, D), jnp.float32),
        pltpu.VMEM_SHARED((NROW, D), jnp.float32),
        pltpu.SemaphoreType.DMA,
        pltpu.SemaphoreType.DMA,
        pltpu.SemaphoreType.DMA,
        pltpu.SemaphoreType.DMA,
        pltpu.SemaphoreType.DMA,
        pltpu.SemaphoreType.DMA,
        pltpu.SemaphoreType.DMA,
        pltpu.SemaphoreType.DMA,
    ],
)(_agg_body)


def kernel(x, edge_index, W1, b1, W2, b2, Wout, bout):
    pad = EPAD - E
    src = jnp.concatenate(
        [edge_index[0], jnp.zeros((pad,), jnp.int32)]).reshape(-1, K)
    dst = jnp.concatenate(
        [edge_index[1], N + (jnp.arange(pad, dtype=jnp.int32) % 128)]
    ).reshape(-1, K)
    t1 = _mm(x, W1)
    accs1 = _agg(t1, src, dst)
    h1, t2 = _epi_mm(accs1, t1, b1.reshape(1, D), W2)
    accs2 = _agg(t2, src, dst)
    return _final(accs2, t2, b2.reshape(1, D), h1, Wout, bout.reshape(1, PE))


# speedup vs baseline: 2.9352x; 2.9352x over previous
import functools

import jax
import jax.numpy as jnp
from jax import lax
from jax.experimental import pallas as pl
from jax.experimental.pallas import tpu as pltpu
from jax.experimental.pallas import tpu_sc as plsc

N = 10000
E = 320000
D = 128
PE = 16
NC = 2
NT = 16
K = 128
NCHUNK = 80
EPT = NCHUNK * K
EPAD = NC * NT * EPT
NROW = N + 128
RCH = 80
NRC = N // RCH

ROW_BLOCK = 2000
GRID = N // ROW_BLOCK


def _mm_body(x_ref, w_ref, t_ref):
    t_ref[...] = jnp.dot(x_ref[...], w_ref[...],
                         preferred_element_type=jnp.float32)


def _mm(x, w):
    return pl.pallas_call(
        _mm_body,
        grid=(GRID,),
        in_specs=[
            pl.BlockSpec((ROW_BLOCK, D), lambda i: (i, 0)),
            pl.BlockSpec((D, D), lambda i: (0, 0)),
        ],
        out_specs=pl.BlockSpec((ROW_BLOCK, D), lambda i: (i, 0)),
        out_shape=jax.ShapeDtypeStruct((N, D), jnp.float32),
    )(x, w)


def _epi_mm_body(a0_ref, a1_ref, t_ref, b_ref, w_ref, h_ref, u_ref):
    h = a0_ref[...] + a1_ref[...] - t_ref[...] + b_ref[...]
    h = jnp.maximum(h, 0.0)
    h_ref[...] = h
    u_ref[...] = jnp.dot(h, w_ref[...], preferred_element_type=jnp.float32)


def _epi_mm(accs, t, b, w):
    return pl.pallas_call(
        _epi_mm_body,
        grid=(GRID,),
        in_specs=[
            pl.BlockSpec((ROW_BLOCK, D), lambda i: (i, 0)),
            pl.BlockSpec((ROW_BLOCK, D), lambda i: (i + GRID, 0)),
            pl.BlockSpec((ROW_BLOCK, D), lambda i: (i, 0)),
            pl.BlockSpec((1, D), lambda i: (0, 0)),
            pl.BlockSpec((D, D), lambda i: (0, 0)),
        ],
        out_specs=[
            pl.BlockSpec((ROW_BLOCK, D), lambda i: (i, 0)),
            pl.BlockSpec((ROW_BLOCK, D), lambda i: (i, 0)),
        ],
        out_shape=[
            jax.ShapeDtypeStruct((N, D), jnp.float32),
            jax.ShapeDtypeStruct((N, D), jnp.float32),
        ],
    )(accs, accs, t, b, w)


def _final_body(a0_ref, a1_ref, t_ref, b_ref, h1_ref, wout_ref, bout_ref,
                out_ref):
    x2 = a0_ref[...] + a1_ref[...] - t_ref[...] + b_ref[...]
    x2 = jnp.maximum(x2, 0.0) + h1_ref[...]
    out_ref[...] = (
        jnp.dot(x2, wout_ref[...], preferred_element_type=jnp.float32)
        + bout_ref[...]
    )


def _final(accs, t, b, h1, wout, bout):
    return pl.pallas_call(
        _final_body,
        grid=(GRID,),
        in_specs=[
            pl.BlockSpec((ROW_BLOCK, D), lambda i: (i, 0)),
            pl.BlockSpec((ROW_BLOCK, D), lambda i: (i + GRID, 0)),
            pl.BlockSpec((ROW_BLOCK, D), lambda i: (i, 0)),
            pl.BlockSpec((1, D), lambda i: (0, 0)),
            pl.BlockSpec((ROW_BLOCK, D), lambda i: (i, 0)),
            pl.BlockSpec((D, PE), lambda i: (0, 0)),
            pl.BlockSpec((1, PE), lambda i: (0, 0)),
        ],
        out_specs=pl.BlockSpec((ROW_BLOCK, PE), lambda i: (i, 0)),
        out_shape=jax.ShapeDtypeStruct((N, PE), jnp.float32),
    )(accs, accs, t, b, h1, wout, bout)


def _agg_body(t_hbm, src_hbm, dst_hbm, accs_out,
              src_b, dst_b, rows_v, acc_sh,
              sem_g0, sem_g1, sem_s0, sem_s1,
              sem_i0, sem_i1, sem_i2, sem_i3):
    c = lax.axis_index("c")
    s = lax.axis_index("s")
    base = (c * NT + s) * NCHUNK
    sems_g = (sem_g0, sem_g1)
    sems_s = (sem_s0, sem_s1)
    sems_i = (sem_i0, sem_i1, sem_i2, sem_i3)
    nmine = (NRC - 1 - s) // NT + 1

    def icopies(j, q):
        return (pltpu.make_async_copy(src_hbm.at[base + j], src_b.at[q],
                                      sems_i[q]),
                pltpu.make_async_copy(dst_hbm.at[base + j], dst_b.at[q],
                                      sems_i[q]))

    def istart(j, q):
        ca, cb = icopies(j, q)
        ca.start()
        cb.start()

    def iwait(j, q):
        ca, cb = icopies(j, q)
        ca.wait()
        cb.wait()

    def gcopy(b, q):
        return pltpu.make_async_copy(
            t_hbm.at[src_b.at[q]], rows_v.at[b], sems_g[b])

    def swait(b, q):
        pltpu.make_async_copy(
            rows_v.at[b], acc_sh.at[dst_b.at[q]], sems_s[b]).wait()

    istart(0, 0)
    istart(1, 1)

    def init_body(q, carry):
        r = (s + q * NT) * RCH
        pltpu.sync_copy(t_hbm.at[pl.ds(r, RCH)], acc_sh.at[pl.ds(r, RCH)])
        return carry

    lax.fori_loop(0, nmine, init_body, 0)
    plsc.subcore_barrier()

    def step(j, r):
        b = r % 2
        gcopy(b, r).wait()
        pltpu.async_copy(rows_v.at[b], acc_sh.at[dst_b.at[r]],
                         sems_s[b], add=True)

        @pl.when(j + 2 < NCHUNK)
        def _():
            istart(j + 2, (r + 2) % 4)

        @pl.when(j >= 1)
        def _():
            swait(1 - b, (r + 3) % 4)

        @pl.when(j + 1 < NCHUNK)
        def _():
            iwait(j + 1, (r + 1) % 4)
            gcopy(1 - b, (r + 1) % 4).start()

    iwait(0, 0)
    gcopy(0, 0).start()

    def loop_body(i, carry):
        for r in range(4):
            step(i * 4 + r, r)
        return carry

    lax.fori_loop(0, NCHUNK // 4, loop_body, 0)
    swait(1, 3)
    plsc.subcore_barrier()

    def drain_body(q, carry):
        r = (s + q * NT) * RCH
        pltpu.sync_copy(acc_sh.at[pl.ds(r, RCH)],
                        accs_out.at[pl.ds(c * N + r, RCH)])
        return carry

    lax.fori_loop(0, nmine, drain_body, 0)


_agg = functools.partial(
    pl.kernel,
    mesh=plsc.VectorSubcoreMesh(core_axis_name="c", subcore_axis_name="s"),
    out_type=jax.ShapeDtypeStruct((NC * N, D), jnp.float32),
    scratch_types=[
        pltpu.VMEM((4, K), jnp.int32),
        pltpu.VMEM((4, K), jnp.int32),
        pltpu.VMEM((2, K, D), jnp.float32),
        pltpu.VMEM_SHARED((NROW, D), jnp.float32),
        pltpu.SemaphoreType.DMA,
        pltpu.SemaphoreType.DMA,
        pltpu.SemaphoreType.DMA,
        pltpu.SemaphoreType.DMA,
        pltpu.SemaphoreType.DMA,
        pltpu.SemaphoreType.DMA,
        pltpu.SemaphoreType.DMA,
        pltpu.SemaphoreType.DMA,
    ],
)(_agg_body)


def kernel(x, edge_index, W1, b1, W2, b2, Wout, bout):
    ppt = EPT - E // (NC * NT)
    lanes = jnp.arange(ppt, dtype=jnp.int32) % 128
    pad_src = jnp.broadcast_to(lanes, (NC * NT, ppt))
    pad_dst = jnp.broadcast_to(N + lanes, (NC * NT, ppt))
    src = jnp.concatenate(
        [edge_index[0].reshape(NC * NT, -1), pad_src], axis=1).reshape(-1, K)
    dst = jnp.concatenate(
        [edge_index[1].reshape(NC * NT, -1), pad_dst], axis=1).reshape(-1, K)
    t1 = _mm(x, W1)
    accs1 = _agg(t1, src, dst)
    h1, t2 = _epi_mm(accs1, t1, b1.reshape(1, D), W2)
    accs2 = _agg(t2, src, dst)
    return _final(accs2, t2, b2.reshape(1, D), h1, Wout, bout.reshape(1, PE))


# speedup vs baseline: 3.5416x; 1.2066x over previous
import functools

import jax
import jax.numpy as jnp
from jax import lax
from jax.experimental import pallas as pl
from jax.experimental.pallas import tpu as pltpu
from jax.experimental.pallas import tpu_sc as plsc

N = 10000
E = 320000
D = 128
PE = 16
NC = 2
NT = 16
K = 80
NCHUNK = 128
GROUPS = NCHUNK // 8
EPT = NCHUNK * K
EPAD = NC * NT * EPT
NROW = N + 128
RING = 4
RCH = 80
NRC = N // RCH

ROW_BLOCK = 2000
GRID = N // ROW_BLOCK


def _mm_body(x_ref, w_ref, t_ref):
    t_ref[...] = jnp.dot(x_ref[...], w_ref[...],
                         preferred_element_type=jnp.float32)


def _mm(x, w):
    return pl.pallas_call(
        _mm_body,
        grid=(GRID,),
        in_specs=[
            pl.BlockSpec((ROW_BLOCK, D), lambda i: (i, 0)),
            pl.BlockSpec((D, D), lambda i: (0, 0)),
        ],
        out_specs=pl.BlockSpec((ROW_BLOCK, D), lambda i: (i, 0)),
        out_shape=jax.ShapeDtypeStruct((N, D), jnp.float32),
    )(x, w)


def _epi_mm_body(a0_ref, a1_ref, t_ref, b_ref, w_ref, h_ref, u_ref):
    h = a0_ref[...] + a1_ref[...] - t_ref[...] + b_ref[...]
    h = jnp.maximum(h, 0.0)
    h_ref[...] = h
    u_ref[...] = jnp.dot(h, w_ref[...], preferred_element_type=jnp.float32)


def _epi_mm(accs, t, b, w):
    return pl.pallas_call(
        _epi_mm_body,
        grid=(GRID,),
        in_specs=[
            pl.BlockSpec((ROW_BLOCK, D), lambda i: (i, 0)),
            pl.BlockSpec((ROW_BLOCK, D), lambda i: (i + GRID, 0)),
            pl.BlockSpec((ROW_BLOCK, D), lambda i: (i, 0)),
            pl.BlockSpec((1, D), lambda i: (0, 0)),
            pl.BlockSpec((D, D), lambda i: (0, 0)),
        ],
        out_specs=[
            pl.BlockSpec((ROW_BLOCK, D), lambda i: (i, 0)),
            pl.BlockSpec((ROW_BLOCK, D), lambda i: (i, 0)),
        ],
        out_shape=[
            jax.ShapeDtypeStruct((N, D), jnp.float32),
            jax.ShapeDtypeStruct((N, D), jnp.float32),
        ],
    )(accs, accs, t, b, w)


def _final_body(a0_ref, a1_ref, t_ref, b_ref, h1_ref, wout_ref, bout_ref,
                out_ref):
    x2 = a0_ref[...] + a1_ref[...] - t_ref[...] + b_ref[...]
    x2 = jnp.maximum(x2, 0.0) + h1_ref[...]
    out_ref[...] = (
        jnp.dot(x2, wout_ref[...], preferred_element_type=jnp.float32)
        + bout_ref[...]
    )


def _final(accs, t, b, h1, wout, bout):
    return pl.pallas_call(
        _final_body,
        grid=(GRID,),
        in_specs=[
            pl.BlockSpec((ROW_BLOCK, D), lambda i: (i, 0)),
            pl.BlockSpec((ROW_BLOCK, D), lambda i: (i + GRID, 0)),
            pl.BlockSpec((ROW_BLOCK, D), lambda i: (i, 0)),
            pl.BlockSpec((1, D), lambda i: (0, 0)),
            pl.BlockSpec((ROW_BLOCK, D), lambda i: (i, 0)),
            pl.BlockSpec((D, PE), lambda i: (0, 0)),
            pl.BlockSpec((1, PE), lambda i: (0, 0)),
        ],
        out_specs=pl.BlockSpec((ROW_BLOCK, PE), lambda i: (i, 0)),
        out_shape=jax.ShapeDtypeStruct((N, PE), jnp.float32),
    )(accs, accs, t, b, h1, wout, bout)


def _agg_body(t_hbm, src_hbm, dst_hbm, accs_out,
              src_b, dst_b, rows_v, acc_sh,
              sem_g0, sem_g1, sem_g2, sem_g3,
              sem_s0, sem_s1, sem_s2, sem_s3,
              sem_i0, sem_i1, sem_io):
    c = lax.axis_index("c")
    s = lax.axis_index("s")
    gbase = (c * NT + s) * GROUPS
    sems_g = (sem_g0, sem_g1, sem_g2, sem_g3)
    sems_s = (sem_s0, sem_s1, sem_s2, sem_s3)
    sems_i = (sem_i0, sem_i1)
    nmine = (NRC - 1 - s) // NT + 1

    def icopies(g, gs):
        return (pltpu.make_async_copy(src_hbm.at[gbase + g], src_b.at[gs],
                                      sems_i[gs]),
                pltpu.make_async_copy(dst_hbm.at[gbase + g], dst_b.at[gs],
                                      sems_i[gs]))

    def istart(g, gs):
        ca, cb = icopies(g, gs)
        ca.start()
        cb.start()

    def iwait(g, gs):
        ca, cb = icopies(g, gs)
        ca.wait()
        cb.wait()

    def gcopy(q, gs, jj):
        return pltpu.make_async_copy(
            t_hbm.at[src_b.at[gs, jj]], rows_v.at[q], sems_g[q])

    def sstart(q, gs, jj):
        pltpu.async_copy(rows_v.at[q], acc_sh.at[dst_b.at[gs, jj]],
                         sems_s[q], add=True)

    def swait(q, gs, jj):
        pltpu.make_async_copy(rows_v.at[q], acc_sh.at[dst_b.at[gs, jj]],
                              sems_s[q]).wait()

    istart(0, 0)

    def init_start(q, carry):
        r = (s + q * NT) * RCH
        pltpu.async_copy(t_hbm.at[pl.ds(r, RCH)], acc_sh.at[pl.ds(r, RCH)],
                         sem_io)
        return carry

    def init_wait(q, carry):
        r = (s + q * NT) * RCH
        pltpu.make_async_copy(t_hbm.at[pl.ds(r, RCH)],
                              acc_sh.at[pl.ds(r, RCH)], sem_io).wait()
        return carry

    lax.fori_loop(0, nmine, init_start, 0)
    lax.fori_loop(0, nmine, init_wait, 0)
    plsc.subcore_barrier()

    def step(j, r):
        q, gs, jj = r % RING, (r // 8) % 2, r % 8
        gcopy(q, gs, jj).wait()
        sstart(q, gs, jj)

        @pl.when(j >= 1)
        def _():
            rm = (r - 1) % 16
            swait(rm % RING, (rm // 8) % 2, rm % 8)

        if r % 8 == 0:
            @pl.when(j + 8 < NCHUNK)
            def _():
                istart(j // 8 + 1, ((r // 8) + 1) % 2)

        @pl.when(j + 3 < NCHUNK)
        def _():
            rp = (r + 3) % 16
            if (r + 3) % 8 == 0:
                iwait((j + 3) // 8, ((r + 3) // 8) % 2)
            gcopy(rp % RING, ((r + 3) // 8) % 2, rp % 8).start()

    iwait(0, 0)
    gcopy(0, 0, 0).start()
    gcopy(1, 0, 1).start()
    gcopy(2, 0, 2).start()

    def loop_body(i, carry):
        for r in range(16):
            step(i * 16 + r, r)
        return carry

    lax.fori_loop(0, NCHUNK // 16, loop_body, 0)
    swait(15 % RING, 1, 7)
    plsc.subcore_barrier()

    def drain_start(q, carry):
        r = (s + q * NT) * RCH
        pltpu.async_copy(acc_sh.at[pl.ds(r, RCH)],
                         accs_out.at[pl.ds(c * N + r, RCH)], sem_io)
        return carry

    def drain_wait(q, carry):
        r = (s + q * NT) * RCH
        pltpu.make_async_copy(acc_sh.at[pl.ds(r, RCH)],
                              accs_out.at[pl.ds(c * N + r, RCH)],
                              sem_io).wait()
        return carry

    lax.fori_loop(0, nmine, drain_start, 0)
    lax.fori_loop(0, nmine, drain_wait, 0)


_agg = functools.partial(
    pl.kernel,
    mesh=plsc.VectorSubcoreMesh(core_axis_name="c", subcore_axis_name="s"),
    out_type=jax.ShapeDtypeStruct((NC * N, D), jnp.float32),
    scratch_types=[
        pltpu.VMEM((2, 8, K), jnp.int32),
        pltpu.VMEM((2, 8, K), jnp.int32),
        pltpu.VMEM((RING, K, D), jnp.float32),
        pltpu.VMEM_SHARED((NROW, D), jnp.float32),
        pltpu.SemaphoreType.DMA,
        pltpu.SemaphoreType.DMA,
        pltpu.SemaphoreType.DMA,
        pltpu.SemaphoreType.DMA,
        pltpu.SemaphoreType.DMA,
        pltpu.SemaphoreType.DMA,
        pltpu.SemaphoreType.DMA,
        pltpu.SemaphoreType.DMA,
        pltpu.SemaphoreType.DMA,
        pltpu.SemaphoreType.DMA,
        pltpu.SemaphoreType.DMA,
    ],
)(_agg_body)


def kernel(x, edge_index, W1, b1, W2, b2, Wout, bout):
    ppt = EPT - E // (NC * NT)
    lanes = jnp.arange(ppt, dtype=jnp.int32) % 128
    pad_src = jnp.broadcast_to(lanes, (NC * NT, ppt))
    pad_dst = jnp.broadcast_to(N + lanes, (NC * NT, ppt))
    src = jnp.concatenate(
        [edge_index[0].reshape(NC * NT, -1), pad_src], axis=1).reshape(-1, 8, K)
    dst = jnp.concatenate(
        [edge_index[1].reshape(NC * NT, -1), pad_dst], axis=1).reshape(-1, 8, K)
    t1 = _mm(x, W1)
    accs1 = _agg(t1, src, dst)
    h1, t2 = _epi_mm(accs1, t1, b1.reshape(1, D), W2)
    accs2 = _agg(t2, src, dst)
    return _final(accs2, t2, b2.reshape(1, D), h1, Wout, bout.reshape(1, PE))


# speedup vs baseline: 3.6266x; 1.0240x over previous
import functools

import jax
import jax.numpy as jnp
from jax import lax
from jax.experimental import pallas as pl
from jax.experimental.pallas import tpu as pltpu
from jax.experimental.pallas import tpu_sc as plsc

N = 10000
E = 320000
D = 128
PE = 16
NC = 2
NT = 16
K = 80
NCHUNK = 128
GROUPS = NCHUNK // 8
EPT = NCHUNK * K
EPAD = NC * NT * EPT
NROW = N + 128
RING = 4
RCH = 80
NRC = N // RCH

ROW_BLOCK = 2000
GRID = N // ROW_BLOCK


def _mm_body(x_ref, w_ref, t_ref):
    t_ref[...] = jnp.dot(x_ref[...], w_ref[...],
                         preferred_element_type=jnp.float32)


def _mm(x, w):
    return pl.pallas_call(
        _mm_body,
        grid=(GRID,),
        in_specs=[
            pl.BlockSpec((ROW_BLOCK, D), lambda i: (i, 0)),
            pl.BlockSpec((D, D), lambda i: (0, 0)),
        ],
        out_specs=pl.BlockSpec((ROW_BLOCK, D), lambda i: (i, 0)),
        out_shape=jax.ShapeDtypeStruct((N, D), jnp.float32),
    )(x, w)


def _epi_mm_body(a0_ref, a1_ref, b_ref, w_ref, h_ref, u_ref):
    h = a0_ref[...] + a1_ref[...] + b_ref[...]
    h = jnp.maximum(h, 0.0)
    h_ref[...] = h
    u_ref[...] = jnp.dot(h, w_ref[...], preferred_element_type=jnp.float32)


def _epi_mm(accs, b, w):
    return pl.pallas_call(
        _epi_mm_body,
        grid=(GRID,),
        in_specs=[
            pl.BlockSpec((ROW_BLOCK, D), lambda i: (i, 0)),
            pl.BlockSpec((ROW_BLOCK, D), lambda i: (i + GRID, 0)),
            pl.BlockSpec((1, D), lambda i: (0, 0)),
            pl.BlockSpec((D, D), lambda i: (0, 0)),
        ],
        out_specs=[
            pl.BlockSpec((ROW_BLOCK, D), lambda i: (i, 0)),
            pl.BlockSpec((ROW_BLOCK, D), lambda i: (i, 0)),
        ],
        out_shape=[
            jax.ShapeDtypeStruct((N, D), jnp.float32),
            jax.ShapeDtypeStruct((N, D), jnp.float32),
        ],
    )(accs, accs, b, w)


def _final_body(a0_ref, a1_ref, b_ref, h1_ref, wout_ref, bout_ref,
                out_ref):
    x2 = a0_ref[...] + a1_ref[...] + b_ref[...]
    x2 = jnp.maximum(x2, 0.0) + h1_ref[...]
    out_ref[...] = (
        jnp.dot(x2, wout_ref[...], preferred_element_type=jnp.float32)
        + bout_ref[...]
    )


def _final(accs, b, h1, wout, bout):
    return pl.pallas_call(
        _final_body,
        grid=(GRID,),
        in_specs=[
            pl.BlockSpec((ROW_BLOCK, D), lambda i: (i, 0)),
            pl.BlockSpec((ROW_BLOCK, D), lambda i: (i + GRID, 0)),
            pl.BlockSpec((1, D), lambda i: (0, 0)),
            pl.BlockSpec((ROW_BLOCK, D), lambda i: (i, 0)),
            pl.BlockSpec((D, PE), lambda i: (0, 0)),
            pl.BlockSpec((1, PE), lambda i: (0, 0)),
        ],
        out_specs=pl.BlockSpec((ROW_BLOCK, PE), lambda i: (i, 0)),
        out_shape=jax.ShapeDtypeStruct((N, PE), jnp.float32),
    )(accs, accs, b, h1, wout, bout)


def _agg_body(t_hbm, src_hbm, dst_hbm, accs_out,
              src_b, dst_b, rows_v, acc_sh,
              sem_g0, sem_g1, sem_g2, sem_g3,
              sem_s0, sem_s1, sem_s2, sem_s3,
              sem_i0, sem_i1, sem_io):
    c = lax.axis_index("c")
    s = lax.axis_index("s")
    wid = c * NT + s
    sems_g = (sem_g0, sem_g1, sem_g2, sem_g3)
    sems_s = (sem_s0, sem_s1, sem_s2, sem_s3)
    sems_i = (sem_i0, sem_i1)
    nmine = (NRC - 1 - s) // NT + 1

    def icopies(g, gs):
        r0 = (wid + NC * NT * g) * 8
        return (pltpu.make_async_copy(src_hbm.at[pl.ds(r0, 8)], src_b.at[gs],
                                      sems_i[gs]),
                pltpu.make_async_copy(dst_hbm.at[pl.ds(r0, 8)], dst_b.at[gs],
                                      sems_i[gs]))

    def istart(g, gs):
        ca, cb = icopies(g, gs)
        ca.start()
        cb.start()

    def iwait(g, gs):
        ca, cb = icopies(g, gs)
        ca.wait()
        cb.wait()

    def gcopy(q, gs, jj):
        return pltpu.make_async_copy(
            t_hbm.at[src_b.at[gs, jj]], rows_v.at[q], sems_g[q])

    def sstart(q, gs, jj):
        pltpu.async_copy(rows_v.at[q], acc_sh.at[dst_b.at[gs, jj]],
                         sems_s[q], add=True)

    def swait(q, gs, jj):
        pltpu.make_async_copy(rows_v.at[q], acc_sh.at[dst_b.at[gs, jj]],
                              sems_s[q]).wait()

    istart(0, 0)

    def init_loops(src_of_r, start):
        def body(q, carry):
            r = (s + q * NT) * RCH
            cp = pltpu.make_async_copy(src_of_r(r), acc_sh.at[pl.ds(r, RCH)],
                                       sem_io)
            cp.start() if start else cp.wait()
            return carry

        lax.fori_loop(0, nmine, body, 0)

    @pl.when(c == 0)
    def _():
        init_loops(lambda r: t_hbm.at[pl.ds(r, RCH)], True)
        init_loops(lambda r: t_hbm.at[pl.ds(r, RCH)], False)

    @pl.when(c == 1)
    def _():
        zeros16 = jnp.zeros((16,), jnp.float32)

        def zfill(rr, carry):
            for cc in range(8):
                rows_v[3, rr, pl.ds(cc * 16, 16)] = zeros16
            return carry

        lax.fori_loop(0, K, zfill, 0)
        init_loops(lambda r: rows_v.at[3], True)
        init_loops(lambda r: rows_v.at[3], False)

    plsc.subcore_barrier()

    def step(j, r):
        q, gs, jj = r % RING, (r // 8) % 2, r % 8
        gcopy(q, gs, jj).wait()
        sstart(q, gs, jj)

        @pl.when(j >= 1)
        def _():
            rm = (r - 1) % 16
            swait(rm % RING, (rm // 8) % 2, rm % 8)

        if r % 8 == 0:
            @pl.when(j + 8 < NCHUNK)
            def _():
                istart(j // 8 + 1, ((r // 8) + 1) % 2)

        @pl.when(j + 3 < NCHUNK)
        def _():
            rp = (r + 3) % 16
            if (r + 3) % 8 == 0:
                iwait((j + 3) // 8, ((r + 3) // 8) % 2)
            gcopy(rp % RING, ((r + 3) // 8) % 2, rp % 8).start()

    iwait(0, 0)
    gcopy(0, 0, 0).start()
    gcopy(1, 0, 1).start()
    gcopy(2, 0, 2).start()

    def loop_body(i, carry):
        for r in range(16):
            step(i * 16 + r, r)
        return carry

    lax.fori_loop(0, NCHUNK // 16, loop_body, 0)
    swait(15 % RING, 1, 7)
    plsc.subcore_barrier()

    def drain_start(q, carry):
        r = (s + q * NT) * RCH
        pltpu.async_copy(acc_sh.at[pl.ds(r, RCH)],
                         accs_out.at[pl.ds(c * N + r, RCH)], sem_io)
        return carry

    def drain_wait(q, carry):
        r = (s + q * NT) * RCH
        pltpu.make_async_copy(acc_sh.at[pl.ds(r, RCH)],
                              accs_out.at[pl.ds(c * N + r, RCH)],
                              sem_io).wait()
        return carry

    lax.fori_loop(0, nmine, drain_start, 0)
    lax.fori_loop(0, nmine, drain_wait, 0)


_agg = functools.partial(
    pl.kernel,
    mesh=plsc.VectorSubcoreMesh(core_axis_name="c", subcore_axis_name="s"),
    out_type=jax.ShapeDtypeStruct((NC * N, D), jnp.float32),
    scratch_types=[
        pltpu.VMEM((2, 8, K), jnp.int32),
        pltpu.VMEM((2, 8, K), jnp.int32),
        pltpu.VMEM((RING, K, D), jnp.float32),
        pltpu.VMEM_SHARED((NROW, D), jnp.float32),
        pltpu.SemaphoreType.DMA,
        pltpu.SemaphoreType.DMA,
        pltpu.SemaphoreType.DMA,
        pltpu.SemaphoreType.DMA,
        pltpu.SemaphoreType.DMA,
        pltpu.SemaphoreType.DMA,
        pltpu.SemaphoreType.DMA,
        pltpu.SemaphoreType.DMA,
        pltpu.SemaphoreType.DMA,
        pltpu.SemaphoreType.DMA,
        pltpu.SemaphoreType.DMA,
    ],
)(_agg_body)


def kernel(x, edge_index, W1, b1, W2, b2, Wout, bout):
    pad = EPAD - E
    ar = jnp.arange(pad, dtype=jnp.int32)
    src = jnp.concatenate([edge_index[0], ar % 8192]).reshape(-1, K)
    dst = jnp.concatenate([edge_index[1], N + ar % 128]).reshape(-1, K)
    t1 = _mm(x, W1)
    accs1 = _agg(t1, src, dst)
    h1, t2 = _epi_mm(accs1, b1.reshape(1, D), W2)
    accs2 = _agg(t2, src, dst)
    return _final(accs2, b2.reshape(1, D), h1, Wout, bout.reshape(1, PE))


# speedup vs baseline: 4.0022x; 1.1035x over previous
import functools

import jax
import jax.numpy as jnp
from jax import lax
from jax.experimental import pallas as pl
from jax.experimental.pallas import tpu as pltpu
from jax.experimental.pallas import tpu_sc as plsc

N = 10000
E = 320000
D = 128
PE = 16
NC = 2
NT = 16
K = 128
NCHUNK = 80
NREAL = E // K
NPADC = NC * NT * NCHUNK - NREAL
DUMMY = 96
NROW = N + DUMMY
RING = 3
IRING = 4
UNROLL = 12
NSTEADY = (NCHUNK // UNROLL) * UNROLL
RCH = 80
NRC = N // RCH

ROW_BLOCK = 2000
GRID = N // ROW_BLOCK


def _mm_body(x_ref, w_ref, t_ref):
    t_ref[...] = jnp.dot(x_ref[...], w_ref[...],
                         preferred_element_type=jnp.float32)


def _mm(x, w):
    return pl.pallas_call(
        _mm_body,
        grid=(GRID,),
        in_specs=[
            pl.BlockSpec((ROW_BLOCK, D), lambda i: (i, 0)),
            pl.BlockSpec((D, D), lambda i: (0, 0)),
        ],
        out_specs=pl.BlockSpec((ROW_BLOCK, D), lambda i: (i, 0)),
        out_shape=jax.ShapeDtypeStruct((N, D), jnp.float32),
    )(x, w)


def _epi_mm_body(a0_ref, a1_ref, b_ref, w_ref, h_ref, u_ref):
    h = a0_ref[...] + a1_ref[...] + b_ref[...]
    h = jnp.maximum(h, 0.0)
    h_ref[...] = h
    u_ref[...] = jnp.dot(h, w_ref[...], preferred_element_type=jnp.float32)


def _epi_mm(accs, b, w):
    return pl.pallas_call(
        _epi_mm_body,
        grid=(GRID,),
        in_specs=[
            pl.BlockSpec((ROW_BLOCK, D), lambda i: (i, 0)),
            pl.BlockSpec((ROW_BLOCK, D), lambda i: (i + GRID, 0)),
            pl.BlockSpec((1, D), lambda i: (0, 0)),
            pl.BlockSpec((D, D), lambda i: (0, 0)),
        ],
        out_specs=[
            pl.BlockSpec((ROW_BLOCK, D), lambda i: (i, 0)),
            pl.BlockSpec((ROW_BLOCK, D), lambda i: (i, 0)),
        ],
        out_shape=[
            jax.ShapeDtypeStruct((N, D), jnp.float32),
            jax.ShapeDtypeStruct((N, D), jnp.float32),
        ],
    )(accs, accs, b, w)


def _final_body(a0_ref, a1_ref, b_ref, h1_ref, wout_ref, bout_ref,
                out_ref):
    x2 = a0_ref[...] + a1_ref[...] + b_ref[...]
    x2 = jnp.maximum(x2, 0.0) + h1_ref[...]
    out_ref[...] = (
        jnp.dot(x2, wout_ref[...], preferred_element_type=jnp.float32)
        + bout_ref[...]
    )


def _final(accs, b, h1, wout, bout):
    return pl.pallas_call(
        _final_body,
        grid=(GRID,),
        in_specs=[
            pl.BlockSpec((ROW_BLOCK, D), lambda i: (i, 0)),
            pl.BlockSpec((ROW_BLOCK, D), lambda i: (i + GRID, 0)),
            pl.BlockSpec((1, D), lambda i: (0, 0)),
            pl.BlockSpec((ROW_BLOCK, D), lambda i: (i, 0)),
            pl.BlockSpec((D, PE), lambda i: (0, 0)),
            pl.BlockSpec((1, PE), lambda i: (0, 0)),
        ],
        out_specs=pl.BlockSpec((ROW_BLOCK, PE), lambda i: (i, 0)),
        out_shape=jax.ShapeDtypeStruct((N, PE), jnp.float32),
    )(accs, accs, b, h1, wout, bout)


def _agg_body(t_hbm, edge_hbm, pad_hbm, accs_out,
              idx_b, rows_v, acc_sh,
              sem_g0, sem_g1, sem_g2,
              sem_s0, sem_s1, sem_s2,
              sem_i0, sem_i1, sem_i2, sem_i3, sem_io):
    c = lax.axis_index("c")
    s = lax.axis_index("s")
    wid = c * NT + s
    sems_g = (sem_g0, sem_g1, sem_g2)
    sems_s = (sem_s0, sem_s1, sem_s2)
    sems_i = (sem_i0, sem_i1, sem_i2, sem_i3)
    nmine = (NRC - 1 - s) // NT + 1

    def istart(j, q, real_only=True):
        gch = wid + NC * NT * j
        dst = idx_b.at[pl.ds(2 * q, 2)]
        if real_only:
            off = pl.multiple_of(gch * K, K)
            pltpu.async_copy(edge_hbm.at[:, pl.ds(off, K)], dst, sems_i[q])
        else:
            @pl.when(gch < NREAL)
            def _():
                off = pl.multiple_of(gch * K, K)
                pltpu.async_copy(edge_hbm.at[:, pl.ds(off, K)], dst,
                                 sems_i[q])

            @pl.when(gch >= NREAL)
            def _():
                off = pl.multiple_of((gch - NREAL) * K, K)
                pltpu.async_copy(pad_hbm.at[:, pl.ds(off, K)], dst,
                                 sems_i[q])

    def iwait(q):
        pltpu.make_async_copy(edge_hbm.at[:, pl.ds(0, K)],
                              idx_b.at[pl.ds(2 * q, 2)], sems_i[q]).wait()

    def gcopy(q3, q):
        return pltpu.make_async_copy(
            t_hbm.at[idx_b.at[2 * q]], rows_v.at[q3], sems_g[q3])

    def sstart(q3, q):
        pltpu.async_copy(rows_v.at[q3], acc_sh.at[idx_b.at[2 * q + 1]],
                         sems_s[q3], add=True)

    def swait(q3, q):
        pltpu.make_async_copy(rows_v.at[q3], acc_sh.at[idx_b.at[2 * q + 1]],
                              sems_s[q3]).wait()

    istart(0, 0)
    istart(1, 1)
    istart(2, 2)

    def init_loops(src_of_r, start):
        def body(q, carry):
            r = (s + q * NT) * RCH
            cp = pltpu.make_async_copy(src_of_r(r), acc_sh.at[pl.ds(r, RCH)],
                                       sem_io)
            cp.start() if start else cp.wait()
            return carry

        lax.fori_loop(0, nmine, body, 0)

    @pl.when(c == 0)
    def _():
        init_loops(lambda r: t_hbm.at[pl.ds(r, RCH)], True)
        init_loops(lambda r: t_hbm.at[pl.ds(r, RCH)], False)

    @pl.when(c == 1)
    def _():
        zeros16 = jnp.zeros((16,), jnp.float32)

        def zfill(rr, carry):
            for cc in range(8):
                rows_v[2, rr, pl.ds(cc * 16, 16)] = zeros16
            return carry

        lax.fori_loop(0, RCH, zfill, 0)
        init_loops(lambda r: rows_v.at[2].at[pl.ds(0, RCH)], True)
        init_loops(lambda r: rows_v.at[2].at[pl.ds(0, RCH)], False)

    plsc.subcore_barrier()

    def step(j, r, jstatic):
        q3, q = r % RING, r % IRING
        gcopy(q3, q).wait()
        sstart(q3, q)

        def retire():
            swait((r - 1) % RING, (r - 1) % IRING)

        if jstatic:
            if j >= 1:
                retire()
            if j + 3 < NCHUNK:
                istart(j + 3, (r + 3) % IRING, real_only=False)
            if j + 2 < NCHUNK:
                iwait((r + 2) % IRING)
                gcopy((r + 2) % RING, (r + 2) % IRING).start()
        else:
            if r == 0:
                @pl.when(j >= 1)
                def _():
                    retire()
            else:
                retire()
            istart(j + 3, (r + 3) % IRING)
            iwait((r + 2) % IRING)
            gcopy((r + 2) % RING, (r + 2) % IRING).start()

    iwait(0)
    gcopy(0, 0).start()
    iwait(1)
    gcopy(1, 1).start()

    def loop_body(i, carry):
        for r in range(UNROLL):
            step(i * UNROLL + r, r, False)
        return carry

    lax.fori_loop(0, NSTEADY // UNROLL, loop_body, 0)
    for j in range(NSTEADY, NCHUNK):
        step(j, j % UNROLL, True)
    swait((NCHUNK - 1) % RING, (NCHUNK - 1) % IRING)
    plsc.subcore_barrier()

    def drain_start(q, carry):
        r = (s + q * NT) * RCH
        pltpu.async_copy(acc_sh.at[pl.ds(r, RCH)],
                         accs_out.at[pl.ds(c * N + r, RCH)], sem_io)
        return carry

    def drain_wait(q, carry):
        r = (s + q * NT) * RCH
        pltpu.make_async_copy(acc_sh.at[pl.ds(r, RCH)],
                              accs_out.at[pl.ds(c * N + r, RCH)],
                              sem_io).wait()
        return carry

    lax.fori_loop(0, nmine, drain_start, 0)
    lax.fori_loop(0, nmine, drain_wait, 0)


_agg = functools.partial(
    pl.kernel,
    mesh=plsc.VectorSubcoreMesh(core_axis_name="c", subcore_axis_name="s"),
    out_type=jax.ShapeDtypeStruct((NC * N, D), jnp.float32),
    scratch_types=[
        pltpu.VMEM((2 * IRING, K), jnp.int32),
        pltpu.VMEM((RING, K, D), jnp.float32),
        pltpu.VMEM_SHARED((NROW, D), jnp.float32),
        pltpu.SemaphoreType.DMA,
        pltpu.SemaphoreType.DMA,
        pltpu.SemaphoreType.DMA,
        pltpu.SemaphoreType.DMA,
        pltpu.SemaphoreType.DMA,
        pltpu.SemaphoreType.DMA,
        pltpu.SemaphoreType.DMA,
        pltpu.SemaphoreType.DMA,
        pltpu.SemaphoreType.DMA,
        pltpu.SemaphoreType.DMA,
        pltpu.SemaphoreType.DMA,
    ],
)(_agg_body)


def kernel(x, edge_index, W1, b1, W2, b2, Wout, bout):
    ar = jnp.arange(NPADC * K, dtype=jnp.int32)
    pad_ed = jnp.stack([ar % 8192, N + ar % DUMMY])
    t1 = _mm(x, W1)
    accs1 = _agg(t1, edge_index, pad_ed)
    h1, t2 = _epi_mm(accs1, b1.reshape(1, D), W2)
    accs2 = _agg(t2, edge_index, pad_ed)
    return _final(accs2, b2.reshape(1, D), h1, Wout, bout.reshape(1, PE))


# speedup vs baseline: 4.2167x; 1.0536x over previous
import functools

import jax
import jax.numpy as jnp
from jax import lax
from jax.experimental import pallas as pl
from jax.experimental.pallas import tpu as pltpu
from jax.experimental.pallas import tpu_sc as plsc

N = 10000
E = 320000
D = 128
PE = 16
NC = 2
NT = 16
K = 128
NCHUNK = 80
NREAL = E // K
NPADC = NC * NT * NCHUNK - NREAL
DUMMY = 96
NROW = N + DUMMY
RING = 3
IRING = 4
UNROLL = 12
NSTEADY = (NCHUNK // UNROLL) * UNROLL
RCH = 80
NRC = N // RCH

ROW_BLOCK = 2000
GRID = N // ROW_BLOCK


def _mm_body(x_ref, w_ref, t_ref):
    t_ref[...] = jnp.dot(x_ref[...], w_ref[...],
                         preferred_element_type=jnp.float32)


def _mm(x, w):
    return pl.pallas_call(
        _mm_body,
        grid=(GRID,),
        in_specs=[
            pl.BlockSpec((ROW_BLOCK, D), lambda i: (i, 0)),
            pl.BlockSpec((D, D), lambda i: (0, 0)),
        ],
        out_specs=pl.BlockSpec((ROW_BLOCK, D), lambda i: (i, 0)),
        out_shape=jax.ShapeDtypeStruct((N, D), jnp.float32),
    )(x, w)


def _epi_mm_body(a0_ref, a1_ref, b_ref, w_ref, h_ref, u_ref):
    h = a0_ref[...] + a1_ref[...] + b_ref[...]
    h = jnp.maximum(h, 0.0)
    h_ref[...] = h
    u_ref[...] = jnp.dot(h, w_ref[...], preferred_element_type=jnp.float32)


def _epi_mm(accs, b, w):
    return pl.pallas_call(
        _epi_mm_body,
        grid=(GRID,),
        in_specs=[
            pl.BlockSpec((ROW_BLOCK, D), lambda i: (i, 0)),
            pl.BlockSpec((ROW_BLOCK, D), lambda i: (i + GRID, 0)),
            pl.BlockSpec((1, D), lambda i: (0, 0)),
            pl.BlockSpec((D, D), lambda i: (0, 0)),
        ],
        out_specs=[
            pl.BlockSpec((ROW_BLOCK, D), lambda i: (i, 0)),
            pl.BlockSpec((ROW_BLOCK, D), lambda i: (i, 0)),
        ],
        out_shape=[
            jax.ShapeDtypeStruct((N, D), jnp.float32),
            jax.ShapeDtypeStruct((N, D), jnp.float32),
        ],
    )(accs, accs, b, w)


def _final_body(a0_ref, a1_ref, b_ref, h1_ref, wout_ref, bout_ref,
                out_ref):
    x2 = a0_ref[...] + a1_ref[...] + b_ref[...]
    x2 = jnp.maximum(x2, 0.0) + h1_ref[...]
    out_ref[...] = (
        jnp.dot(x2, wout_ref[...], preferred_element_type=jnp.float32)
        + bout_ref[...]
    )


def _final(accs, b, h1, wout, bout):
    return pl.pallas_call(
        _final_body,
        grid=(GRID,),
        in_specs=[
            pl.BlockSpec((ROW_BLOCK, D), lambda i: (i, 0)),
            pl.BlockSpec((ROW_BLOCK, D), lambda i: (i + GRID, 0)),
            pl.BlockSpec((1, D), lambda i: (0, 0)),
            pl.BlockSpec((ROW_BLOCK, D), lambda i: (i, 0)),
            pl.BlockSpec((D, PE), lambda i: (0, 0)),
            pl.BlockSpec((1, PE), lambda i: (0, 0)),
        ],
        out_specs=pl.BlockSpec((ROW_BLOCK, PE), lambda i: (i, 0)),
        out_shape=jax.ShapeDtypeStruct((N, PE), jnp.float32),
    )(accs, accs, b, h1, wout, bout)


def _agg_body(t_hbm, edge_hbm, pad_hbm, accs_out,
              idx_b, rows_v, acc_sh,
              sem_g0, sem_g1, sem_g2,
              sem_s0, sem_s1, sem_s2,
              sem_i0, sem_i1, sem_i2, sem_i3, sem_io):
    c = lax.axis_index("c")
    s = lax.axis_index("s")
    wid = c * NT + s
    sems_g = (sem_g0, sem_g1, sem_g2)
    sems_s = (sem_s0, sem_s1, sem_s2)
    sems_i = (sem_i0, sem_i1, sem_i2, sem_i3)
    nmine = (NRC - 1 - s) // NT + 1

    def istart(j, q, real_only=True):
        gch = wid + NC * NT * j
        dst = idx_b.at[pl.ds(2 * q, 2)]
        if real_only:
            off = pl.multiple_of(gch * K, K)
            pltpu.async_copy(edge_hbm.at[:, pl.ds(off, K)], dst, sems_i[q])
        else:
            @pl.when(gch < NREAL)
            def _():
                off = pl.multiple_of(gch * K, K)
                pltpu.async_copy(edge_hbm.at[:, pl.ds(off, K)], dst,
                                 sems_i[q])

            @pl.when(gch >= NREAL)
            def _():
                off = pl.multiple_of((gch - NREAL) * K, K)
                pltpu.async_copy(pad_hbm.at[:, pl.ds(off, K)], dst,
                                 sems_i[q])

    def iwait(q):
        pltpu.make_async_copy(edge_hbm.at[:, pl.ds(0, K)],
                              idx_b.at[pl.ds(2 * q, 2)], sems_i[q]).wait()

    def gcopy(q3, q):
        return pltpu.make_async_copy(
            t_hbm.at[idx_b.at[2 * q]], rows_v.at[q3], sems_g[q3])

    def sstart(q3, q):
        pltpu.async_copy(rows_v.at[q3], acc_sh.at[idx_b.at[2 * q + 1]],
                         sems_s[q3], add=True)

    def swait(q3, q):
        pltpu.make_async_copy(rows_v.at[q3], acc_sh.at[idx_b.at[2 * q + 1]],
                              sems_s[q3]).wait()

    istart(0, 0)
    istart(1, 1)
    istart(2, 2)
    iwait(0)
    gcopy(0, 0).start()
    iwait(1)
    gcopy(1, 1).start()

    def init_loops(src_of_r, start):
        def body(q, carry):
            r = (s + q * NT) * RCH
            cp = pltpu.make_async_copy(src_of_r(r), acc_sh.at[pl.ds(r, RCH)],
                                       sem_io)
            cp.start() if start else cp.wait()
            return carry

        lax.fori_loop(0, nmine, body, 0)

    @pl.when(c == 0)
    def _():
        init_loops(lambda r: t_hbm.at[pl.ds(r, RCH)], True)
        init_loops(lambda r: t_hbm.at[pl.ds(r, RCH)], False)

    @pl.when(c == 1)
    def _():
        zeros16 = jnp.zeros((16,), jnp.float32)

        def zfill(rr, carry):
            for cc in range(8):
                rows_v[2, rr, pl.ds(cc * 16, 16)] = zeros16
            return carry

        lax.fori_loop(0, RCH, zfill, 0)
        init_loops(lambda r: rows_v.at[2].at[pl.ds(0, RCH)], True)
        init_loops(lambda r: rows_v.at[2].at[pl.ds(0, RCH)], False)

    plsc.subcore_barrier()

    def step(j, r, jstatic):
        q3, q = r % RING, r % IRING

        def retire():
            swait((r - 1) % RING, (r - 1) % IRING)

        if jstatic:
            if j >= 1:
                retire()
            if j + 3 < NCHUNK:
                istart(j + 3, (r + 3) % IRING, real_only=False)
            if j + 2 < NCHUNK:
                iwait((r + 2) % IRING)
                gcopy((r + 2) % RING, (r + 2) % IRING).start()
        else:
            if r == 0:
                @pl.when(j >= 1)
                def _():
                    retire()
            else:
                retire()
            istart(j + 3, (r + 3) % IRING)
            iwait((r + 2) % IRING)
            gcopy((r + 2) % RING, (r + 2) % IRING).start()
        gcopy(q3, q).wait()
        sstart(q3, q)

    def loop_body(i, carry):
        for r in range(UNROLL):
            step(i * UNROLL + r, r, False)
        return carry

    lax.fori_loop(0, NSTEADY // UNROLL, loop_body, 0)
    for j in range(NSTEADY, NCHUNK):
        step(j, j % UNROLL, True)
    swait((NCHUNK - 1) % RING, (NCHUNK - 1) % IRING)
    plsc.subcore_barrier()

    def drain_start(q, carry):
        r = (s + q * NT) * RCH
        pltpu.async_copy(acc_sh.at[pl.ds(r, RCH)],
                         accs_out.at[pl.ds(c * N + r, RCH)], sem_io)
        return carry

    def drain_wait(q, carry):
        r = (s + q * NT) * RCH
        pltpu.make_async_copy(acc_sh.at[pl.ds(r, RCH)],
                              accs_out.at[pl.ds(c * N + r, RCH)],
                              sem_io).wait()
        return carry

    lax.fori_loop(0, nmine, drain_start, 0)
    lax.fori_loop(0, nmine, drain_wait, 0)


_agg = functools.partial(
    pl.kernel,
    mesh=plsc.VectorSubcoreMesh(core_axis_name="c", subcore_axis_name="s"),
    out_type=jax.ShapeDtypeStruct((NC * N, D), jnp.float32),
    scratch_types=[
        pltpu.VMEM((2 * IRING, K), jnp.int32),
        pltpu.VMEM((RING, K, D), jnp.float32),
        pltpu.VMEM_SHARED((NROW, D), jnp.float32),
        pltpu.SemaphoreType.DMA,
        pltpu.SemaphoreType.DMA,
        pltpu.SemaphoreType.DMA,
        pltpu.SemaphoreType.DMA,
        pltpu.SemaphoreType.DMA,
        pltpu.SemaphoreType.DMA,
        pltpu.SemaphoreType.DMA,
        pltpu.SemaphoreType.DMA,
        pltpu.SemaphoreType.DMA,
        pltpu.SemaphoreType.DMA,
        pltpu.SemaphoreType.DMA,
    ],
)(_agg_body)


def kernel(x, edge_index, W1, b1, W2, b2, Wout, bout):
    ar = jnp.arange(NPADC * K, dtype=jnp.int32)
    pad_ed = jnp.stack([ar % 8192, N + ar % DUMMY])
    t1 = _mm(x, W1)
    accs1 = _agg(t1, edge_index, pad_ed)
    h1, t2 = _epi_mm(accs1, b1.reshape(1, D), W2)
    accs2 = _agg(t2, edge_index, pad_ed)
    return _final(accs2, b2.reshape(1, D), h1, Wout, bout.reshape(1, PE))
